# baseline (speedup 1.0000x reference)
import functools

import jax
import jax.numpy as jnp
from jax import lax
from jax.experimental import pallas as pl
from jax.experimental.pallas import tpu as pltpu
from jax.experimental.pallas import tpu_sc as plsc

B = 16384
L = 50
D = 16
ZCH_SIZE = 1000000

NW = 32
BAGS_PER_W = B // NW
IDS_PER_W = BAGS_PER_W * L
SUP_BAGS = 16
NSUP = BAGS_PER_W // SUP_BAGS
SUP_IDS = SUP_BAGS * L
GPARTS = [(0, SUP_IDS)]
NBUF = 6


def _build():
    mesh = plsc.VectorSubcoreMesh(core_axis_name="c", subcore_axis_name="s")

    @functools.partial(
        pl.kernel,
        mesh=mesh,
        out_type=jax.ShapeDtypeStruct((B, D), jnp.float32),
        compiler_params=pltpu.CompilerParams(use_tc_tiling_on_sc=False),
        scratch_types=[
            pltpu.VMEM((IDS_PER_W,), jnp.int32),
            pltpu.VMEM((NBUF * SUP_IDS, D), jnp.float32),
            pltpu.VMEM((BAGS_PER_W, D), jnp.float32),
            pltpu.SemaphoreType.DMA,
            pltpu.SemaphoreType.DMA,
            pltpu.SemaphoreType.DMA,
            pltpu.SemaphoreType.DMA,
            pltpu.SemaphoreType.DMA,
            pltpu.SemaphoreType.DMA,
        ],
    )
    def k(vals_hbm, table_hbm, out_hbm, idx_v, rows_v, out_v,
          s0, s1, s2, s3, s4, s5):
        sems = (s0, s1, s2, s3, s4, s5)
        wid = lax.axis_index("s") * 2 + lax.axis_index("c")
        base = wid * IDS_PER_W

        pltpu.sync_copy(vals_hbm.at[pl.ds(base, IDS_PER_W)], idx_v)

        def hash_step(i, _):
            v = idx_v[pl.ds(i * 16, 16)].astype(jnp.uint32)
            h = v * jnp.uint32(2654435761)
            h = h ^ (h >> jnp.uint32(16))
            h = h * jnp.uint32(2246822519)
            h = h ^ (h >> jnp.uint32(13))
            idx_v[pl.ds(i * 16, 16)] = (h % jnp.uint32(ZCH_SIZE)).astype(
                jnp.int32)
            return 0

        def hash_chunk(s):
            lax.fori_loop(s * (SUP_IDS // 16), (s + 1) * (SUP_IDS // 16),
                          hash_step, 0)

        def fire(s):
            rbase = (s % NBUF) * SUP_IDS
            for off, n in GPARTS:
                pltpu.make_async_copy(
                    table_hbm.at[idx_v.at[pl.ds(s * SUP_IDS + off, n)]],
                    rows_v.at[pl.ds(rbase + off, n), :],
                    sems[s % NBUF],
                ).start()

        def drain(s):
            rbase = (s % NBUF) * SUP_IDS
            for off, n in GPARTS:
                pltpu.make_async_copy(
                    table_hbm.at[idx_v.at[pl.ds(s * SUP_IDS + off, n)]],
                    rows_v.at[pl.ds(rbase + off, n), :],
                    sems[s % NBUF],
                ).wait()

        def pool(s):
            rbase = (s % NBUF) * SUP_IDS

            def bag_step(j, _):
                r0 = rbase + j * L
                accs = [rows_v[r0 + a, :] for a in range(5)]
                for l in range(5, L):
                    a = l % 5
                    accs[a] = accs[a] + rows_v[r0 + l, :]
                out_v[s * SUP_BAGS + j, :] = (
                    (accs[0] + accs[1]) + (accs[2] + accs[3]) + accs[4])
                return 0

            lax.fori_loop(0, SUP_BAGS, bag_step, 0)

        AHEAD = NBUF - 1
        for s in range(AHEAD):
            hash_chunk(s)
            fire(s)
        for s in range(NSUP):
            if s + AHEAD < NSUP:
                hash_chunk(s + AHEAD)
                fire(s + AHEAD)
            drain(s)
            pool(s)

        pltpu.sync_copy(out_v, out_hbm.at[pl.ds(wid * BAGS_PER_W, BAGS_PER_W), :])

    return k


_kernel = _build()


@jax.jit
def kernel(values, table):
    return _kernel(values.reshape(-1), table)


# speedup vs baseline: 1.0043x; 1.0043x over previous
import functools

import jax
import jax.numpy as jnp
from jax import lax
from jax.experimental import pallas as pl
from jax.experimental.pallas import tpu as pltpu
from jax.experimental.pallas import tpu_sc as plsc

B = 16384
L = 50
D = 16
ZCH_SIZE = 1000000

NW = 32
BAGS_PER_W = B // NW
IDS_PER_W = BAGS_PER_W * L
SUP_BAGS = 32
NSUP = BAGS_PER_W // SUP_BAGS
SUP_IDS = SUP_BAGS * L
GPARTS = [(0, 1600)]
NBUF = 3


def _build():
    mesh = plsc.VectorSubcoreMesh(core_axis_name="c", subcore_axis_name="s")

    @functools.partial(
        pl.kernel,
        mesh=mesh,
        out_type=jax.ShapeDtypeStruct((B, D), jnp.float32),
        compiler_params=pltpu.CompilerParams(use_tc_tiling_on_sc=False),
        scratch_types=[
            pltpu.VMEM((IDS_PER_W,), jnp.int32),
            pltpu.VMEM((NBUF * SUP_IDS, D), jnp.float32),
            pltpu.VMEM((BAGS_PER_W, D), jnp.float32),
            pltpu.SemaphoreType.DMA,
            pltpu.SemaphoreType.DMA,
            pltpu.SemaphoreType.DMA,
        ],
    )
    def k(vals_hbm, table_hbm, out_hbm, idx_v, rows_v, out_v, s0, s1, s2):
        sems = (s0, s1, s2)
        wid = lax.axis_index("s") * 2 + lax.axis_index("c")
        base = wid * IDS_PER_W

        pltpu.sync_copy(vals_hbm.at[pl.ds(base, IDS_PER_W)], idx_v)

        def hash_step(i, _):
            v = idx_v[pl.ds(i * 16, 16)].astype(jnp.uint32)
            h = v * jnp.uint32(2654435761)
            h = h ^ (h >> jnp.uint32(16))
            h = h * jnp.uint32(2246822519)
            h = h ^ (h >> jnp.uint32(13))
            idx_v[pl.ds(i * 16, 16)] = (h % jnp.uint32(ZCH_SIZE)).astype(
                jnp.int32)
            return 0

        def hash_chunk(s):
            lax.fori_loop(s * (SUP_IDS // 16), (s + 1) * (SUP_IDS // 16),
                          hash_step, 0)

        def fire(s):
            rbase = (s % NBUF) * SUP_IDS
            for off, n in GPARTS:
                pltpu.make_async_copy(
                    table_hbm.at[idx_v.at[pl.ds(s * SUP_IDS + off, n)]],
                    rows_v.at[pl.ds(rbase + off, n), :],
                    sems[s % NBUF],
                ).start()

        def drain(s):
            rbase = (s % NBUF) * SUP_IDS
            for off, n in GPARTS:
                pltpu.make_async_copy(
                    table_hbm.at[idx_v.at[pl.ds(s * SUP_IDS + off, n)]],
                    rows_v.at[pl.ds(rbase + off, n), :],
                    sems[s % NBUF],
                ).wait()

        def pool(s):
            rbase = (s % NBUF) * SUP_IDS

            def bag_step(j, _):
                r0 = rbase + j * L
                accs = [rows_v[r0 + a, :] for a in range(5)]
                for l in range(5, L):
                    a = l % 5
                    accs[a] = accs[a] + rows_v[r0 + l, :]
                out_v[s * SUP_BAGS + j, :] = (
                    (accs[0] + accs[1]) + (accs[2] + accs[3]) + accs[4])
                return 0

            lax.fori_loop(0, SUP_BAGS, bag_step, 0)

        hash_chunk(0)
        fire(0)
        hash_chunk(1)
        fire(1)
        for s in range(NSUP):
            if s + 2 < NSUP:
                hash_chunk(s + 2)
                fire(s + 2)
            drain(s)
            pool(s)

        pltpu.sync_copy(out_v, out_hbm.at[pl.ds(wid * BAGS_PER_W, BAGS_PER_W), :])

    return k


_kernel = _build()


@jax.jit
def kernel(values, table):
    return _kernel(values.reshape(-1), table)
